# Initial kernel scaffold; baseline (speedup 1.0000x reference)
import jax
import jax.numpy as jnp
from jax.experimental import pallas as pl


def kernel(x, edge_index, W1, b1, g1, be1, W2, b2, g2, be2, W3, b3):
    raise NotImplementedError("write your pallas kernel here")


# speedup vs baseline: 12.8137x; 12.8137x over previous
import functools

import jax
import jax.numpy as jnp
from jax import lax
from jax.experimental import pallas as pl
from jax.experimental.pallas import tpu as pltpu
from jax.experimental.pallas import tpu_sc as plsc

_N = 10000
_E = 320000
_D = 128
_NCLS = 40
_DP = 48
_NC = 2
_NS = 16
_NW = _NC * _NS
_K = 80
_NCH = _E // (_NW * _K)
_RPW = _N // _NS
_ZR = 125
_DEGW = 16
_BLK = 200

_mesh = plsc.VectorSubcoreMesh(
    core_axis_name="c", subcore_axis_name="s", num_cores=_NC, num_subcores=_NS
)


def _make_prop(D):
  @functools.partial(
      pl.kernel,
      out_type=jax.ShapeDtypeStruct((_NC, _N, D), jnp.float32),
      mesh=_mesh,
      compiler_params=pltpu.CompilerParams(use_tc_tiling_on_sc=False),
      scratch_types=[
          pltpu.VMEM((_NCH, _K), jnp.int32),
          pltpu.VMEM((_NCH, _K), jnp.int32),
          pltpu.VMEM((_K, D), jnp.float32),
          pltpu.VMEM((_ZR, D), jnp.float32),
          pltpu.VMEM_SHARED((_N, D), jnp.float32),
          pltpu.SemaphoreType.DMA,
      ],
  )
  def prop(h_hbm, src_hbm, dst_hbm, out_hbm, src_v, dst_v, rows_v, z_v,
           acc_sh, sem):
    c = lax.axis_index("c")
    s = lax.axis_index("s")
    w = c * _NS + s
    zero16 = jnp.zeros((16,), jnp.float32)

    @pl.loop(0, _ZR)
    def _zrow(i):
      for d in range(D // 16):
        z_v[i, pl.ds(d * 16, 16)] = zero16

    @pl.loop(0, _RPW // _ZR)
    def _zacc(t):
      pltpu.sync_copy(z_v, acc_sh.at[pl.ds(s * _RPW + t * _ZR, _ZR)])

    plsc.subcore_barrier()

    pltpu.sync_copy(src_hbm.at[w], src_v)
    pltpu.sync_copy(dst_hbm.at[w], dst_v)

    @pl.loop(0, _NCH)
    def _edges(j):
      pltpu.async_copy(h_hbm.at[src_v.at[j]], rows_v, sem).wait()
      pltpu.sync_copy(rows_v, acc_sh.at[dst_v.at[j]], add=True)

    plsc.subcore_barrier()
    pltpu.sync_copy(acc_sh.at[pl.ds(s * _RPW, _RPW)],
                    out_hbm.at[c, pl.ds(s * _RPW, _RPW)])

  return prop


@functools.partial(
    pl.kernel,
    out_type=jax.ShapeDtypeStruct((_NC, _N, _DEGW), jnp.float32),
    mesh=_mesh,
    compiler_params=pltpu.CompilerParams(use_tc_tiling_on_sc=False),
    scratch_types=[
        pltpu.VMEM((_NCH, _K), jnp.int32),
        pltpu.VMEM((_K, _DEGW), jnp.float32),
        pltpu.VMEM((_ZR, _DEGW), jnp.float32),
        pltpu.VMEM_SHARED((_N, _DEGW), jnp.float32),
    ],
)
def _deg_kernel(dst_hbm, out_hbm, dst_v, ones_v, z_v, acc_sh):
  c = lax.axis_index("c")
  s = lax.axis_index("s")
  w = c * _NS + s
  zero16 = jnp.zeros((16,), jnp.float32)
  one16 = jnp.ones((16,), jnp.float32)

  @pl.loop(0, _ZR)
  def _zrow(i):
    z_v[i, pl.ds(0, 16)] = zero16

  @pl.loop(0, _K)
  def _orow(i):
    ones_v[i, pl.ds(0, 16)] = one16

  @pl.loop(0, _RPW // _ZR)
  def _zacc(t):
    pltpu.sync_copy(z_v, acc_sh.at[pl.ds(s * _RPW + t * _ZR, _ZR)])

  plsc.subcore_barrier()

  pltpu.sync_copy(dst_hbm.at[w], dst_v)

  @pl.loop(0, _NCH)
  def _edges(j):
    pltpu.sync_copy(ones_v, acc_sh.at[dst_v.at[j]], add=True)

  plsc.subcore_barrier()
  pltpu.sync_copy(acc_sh.at[pl.ds(s * _RPW, _RPW)],
                  out_hbm.at[c, pl.ds(s * _RPW, _RPW)])


def _first_body(deg_ref, x_ref, w_ref, h_ref, dinv_ref):
  deg = deg_ref[0] + deg_ref[1] + 1.0
  dinv = lax.rsqrt(deg)
  dinv_ref[...] = dinv
  h = jnp.dot(x_ref[...], w_ref[...], preferred_element_type=jnp.float32)
  h_ref[...] = h * dinv[:, 0:1]


def _tc_first(degp, x, w1):
  return pl.pallas_call(
      _first_body,
      grid=(_N // _BLK,),
      in_specs=[
          pl.BlockSpec((_NC, _BLK, _DEGW), lambda i: (0, i, 0)),
          pl.BlockSpec((_BLK, _D), lambda i: (i, 0)),
          pl.BlockSpec((_D, _D), lambda i: (0, 0)),
      ],
      out_specs=[
          pl.BlockSpec((_BLK, _D), lambda i: (i, 0)),
          pl.BlockSpec((_BLK, _DEGW), lambda i: (i, 0)),
      ],
      out_shape=[
          jax.ShapeDtypeStruct((_N, _D), jnp.float32),
          jax.ShapeDtypeStruct((_N, _DEGW), jnp.float32),
      ],
  )(degp, x, w1)


def _mid_body(p_ref, h_ref, dinv_ref, b_ref, g_ref, be_ref, w_ref, o_ref):
  dc = dinv_ref[...][:, 0:1]
  t = (p_ref[0] + p_ref[1] + h_ref[...]) * dc + b_ref[...]
  t = jnp.where(t > 0.0, t, jnp.exp(jnp.minimum(t, 0.0)) - 1.0)
  mu = jnp.mean(t, axis=-1, keepdims=True)
  tcen = t - mu
  var = jnp.mean(tcen * tcen, axis=-1, keepdims=True)
  z = tcen * lax.rsqrt(var + 1e-5) * g_ref[...] + be_ref[...]
  o_ref[...] = jnp.dot(
      z, w_ref[...], preferred_element_type=jnp.float32) * dc


def _tc_mid(p, hprev, dinv, b, g, be, wnext):
  dn = wnext.shape[1]
  return pl.pallas_call(
      _mid_body,
      grid=(_N // _BLK,),
      in_specs=[
          pl.BlockSpec((_NC, _BLK, _D), lambda i: (0, i, 0)),
          pl.BlockSpec((_BLK, _D), lambda i: (i, 0)),
          pl.BlockSpec((_BLK, _DEGW), lambda i: (i, 0)),
          pl.BlockSpec((1, _D), lambda i: (0, 0)),
          pl.BlockSpec((1, _D), lambda i: (0, 0)),
          pl.BlockSpec((1, _D), lambda i: (0, 0)),
          pl.BlockSpec((_D, dn), lambda i: (0, 0)),
      ],
      out_specs=pl.BlockSpec((_BLK, dn), lambda i: (i, 0)),
      out_shape=jax.ShapeDtypeStruct((_N, dn), jnp.float32),
  )(p, hprev, dinv, b.reshape(1, _D), g.reshape(1, _D), be.reshape(1, _D),
    wnext)


def _final_body(p_ref, h_ref, dinv_ref, b_ref, o_ref):
  dc = dinv_ref[...][:, 0:1]
  o_ref[...] = (p_ref[0] + p_ref[1] + h_ref[...]) * dc + b_ref[...]


def _tc_final(p, h3, dinv, b3):
  return pl.pallas_call(
      _final_body,
      grid=(_N // _BLK,),
      in_specs=[
          pl.BlockSpec((_NC, _BLK, _DP), lambda i: (0, i, 0)),
          pl.BlockSpec((_BLK, _DP), lambda i: (i, 0)),
          pl.BlockSpec((_BLK, _DEGW), lambda i: (i, 0)),
          pl.BlockSpec((1, _DP), lambda i: (0, 0)),
      ],
      out_specs=pl.BlockSpec((_BLK, _DP), lambda i: (i, 0)),
      out_shape=jax.ShapeDtypeStruct((_N, _DP), jnp.float32),
  )(p, h3, dinv, b3.reshape(1, _DP))


_prop128 = _make_prop(_D)
_prop48 = _make_prop(_DP)


def kernel(x, edge_index, W1, b1, g1, be1, W2, b2, g2, be2, W3, b3):
  src_r = edge_index[0].reshape(_NW, _NCH, _K)
  dst_r = edge_index[1].reshape(_NW, _NCH, _K)
  w3p = jnp.pad(W3, ((0, 0), (0, _DP - _NCLS)))
  b3p = jnp.pad(b3, (0, _DP - _NCLS))

  degp = _deg_kernel(dst_r)
  h1, dinv = _tc_first(degp, x, W1)
  p1 = _prop128(h1, src_r, dst_r)
  h2 = _tc_mid(p1, h1, dinv, b1, g1, be1, W2)
  p2 = _prop128(h2, src_r, dst_r)
  h3 = _tc_mid(p2, h2, dinv, b2, g2, be2, w3p)
  p3 = _prop48(h3, src_r, dst_r)
  out = _tc_final(p3, h3, dinv, b3p)
  return out[:, :_NCLS]
